# baseline (speedup 1.0000x reference)
import functools

import jax
import jax.numpy as jnp
from jax import lax
from jax.experimental import pallas as pl
from jax.experimental.pallas import tpu as pltpu
from jax.experimental.pallas import tpu_sc as plsc

EMB = 128
HID = 256
OUT = 512
B = 4096
L = 20

NC = 2
NS = 16
NW = NC * NS
LANES = 16


def _sc_gather_pool(category, sty_tl, silhouette, mat_tl, det_tl,
                    cat_t, sty_t, sil_t, mat_t, det_t):
    nb = category.shape[0]
    rows_pt = nb // NW
    pool_pt = nb * L // NW
    mesh = plsc.VectorSubcoreMesh(core_axis_name="c", subcore_axis_name="s")
    out_type = tuple(jax.ShapeDtypeStruct((nb, EMB), jnp.float32)
                     for _ in range(5))

    @functools.partial(
        pl.kernel, mesh=mesh, out_type=out_type,
        scratch_types=[
            pltpu.VMEM((pool_pt,), jnp.int32),
            pltpu.VMEM((pool_pt,), jnp.int32),
            pltpu.VMEM((pool_pt,), jnp.int32),
            pltpu.VMEM((rows_pt,), jnp.int32),
            pltpu.VMEM((rows_pt,), jnp.int32),
            pltpu.VMEM((rows_pt, EMB), jnp.float32),
            pltpu.VMEM((rows_pt, EMB), jnp.float32),
            pltpu.VMEM((rows_pt, EMB), jnp.float32),
            pltpu.VMEM((rows_pt, EMB), jnp.float32),
            pltpu.VMEM((rows_pt, EMB), jnp.float32),
            pltpu.SemaphoreType.DMA,
            pltpu.SemaphoreType.DMA,
            pltpu.SemaphoreType.DMA,
            pltpu.SemaphoreType.DMA,
            pltpu.SemaphoreType.DMA,
            pltpu.SemaphoreType.DMA,
            pltpu.SemaphoreType.DMA,
        ],
    )
    def k(cat_i, sty_i, sil_i, mat_i, det_i,
          cat_th, sty_th, sil_th, mat_th, det_th,
          cat_o, sil_o, sty_o, mat_o, det_o,
          ixp0, ixp1, ixp2, ixs0, ixs1,
          acc0, acc1, acc2, small0, small1,
          sem0, sem1, sem2, sem3, sem4, sem5, sem6):
        wid = lax.axis_index("s") * NC + lax.axis_index("c")
        obase = wid * rows_pt
        gbase = wid * pool_pt

        accs = (acc0, acc1, acc2)
        tabs = (sty_th, mat_th, det_th)
        idxs = (ixp0, ixp1, ixp2)
        sems = (sem0, sem1, sem2)
        outs = (sty_o, mat_o, det_o)

        pltpu.sync_copy(sty_i.at[pl.ds(gbase, pool_pt)], ixp0)
        pltpu.sync_copy(cat_i.at[pl.ds(obase, rows_pt)], ixs0)
        pltpu.sync_copy(sil_i.at[pl.ds(obase, rows_pt)], ixs1)
        pltpu.async_copy(mat_i.at[pl.ds(gbase, pool_pt)], ixp1, sem5)
        pltpu.async_copy(det_i.at[pl.ds(gbase, pool_pt)], ixp2, sem6)

        pltpu.async_copy(cat_th.at[ixs0], small0, sem3)
        pltpu.async_copy(sil_th.at[ixs1], small1, sem4)

        def zero(acc):
            @pl.loop(0, rows_pt)
            def _(r):
                for c in range(EMB // LANES):
                    acc[r, pl.ds(c * LANES, LANES)] = jnp.zeros(
                        (LANES,), jnp.float32)

        def fire(f):
            for l in range(L):
                pltpu.async_copy(
                    tabs[f].at[idxs[f].at[pl.ds(l * rows_pt, rows_pt)]],
                    accs[f], sems[f], add=True)

        zero(acc0)
        fire(0)
        pltpu.make_async_copy(mat_i.at[pl.ds(gbase, pool_pt)], ixp1, sem5).wait()
        zero(acc1)
        fire(1)
        pltpu.make_async_copy(det_i.at[pl.ds(gbase, pool_pt)], ixp2, sem6).wait()
        zero(acc2)
        fire(2)

        for f in range(3):
            for l in range(L):
                pltpu.make_async_copy(
                    tabs[f].at[idxs[f].at[pl.ds(l * rows_pt, rows_pt)]],
                    accs[f], sems[f]).wait()
            pltpu.async_copy(accs[f], outs[f].at[pl.ds(obase, rows_pt)], sems[f])

        pltpu.make_async_copy(cat_th.at[ixs0], small0, sem3).wait()
        pltpu.async_copy(small0, cat_o.at[pl.ds(obase, rows_pt)], sem3)
        pltpu.make_async_copy(sil_th.at[ixs1], small1, sem4).wait()
        pltpu.async_copy(small1, sil_o.at[pl.ds(obase, rows_pt)], sem4)

        for f in range(3):
            pltpu.make_async_copy(accs[f], outs[f].at[pl.ds(obase, rows_pt)],
                                  sems[f]).wait()
        pltpu.make_async_copy(small0, cat_o.at[pl.ds(obase, rows_pt)], sem3).wait()
        pltpu.make_async_copy(small1, sil_o.at[pl.ds(obase, rows_pt)], sem4).wait()

    return k(category, sty_tl, silhouette, mat_tl, det_tl,
             cat_t, sty_t, sil_t, mat_t, det_t)


def _mlp_body(cat_ref, sil_ref, sty_ref, mat_ref, det_ref,
              sm_ref, mm_ref, dm_ref,
              w1_ref, b1_ref, w2_ref, b2_ref, o_ref):
    def pool(sum_ref, m_ref):
        cnt = jnp.maximum(jnp.sum(m_ref[...], axis=1, keepdims=True), 1.0)
        return sum_ref[...] / cnt

    bf = jnp.bfloat16
    sty = pool(sty_ref, sm_ref).astype(bf)
    mat = pool(mat_ref, mm_ref).astype(bf)
    det = pool(det_ref, dm_ref).astype(bf)
    w1 = w1_ref[...].astype(bf)
    h = (jnp.dot(cat_ref[...].astype(bf), w1[0 * EMB:1 * EMB], preferred_element_type=jnp.float32)
         + jnp.dot(sty, w1[1 * EMB:2 * EMB], preferred_element_type=jnp.float32)
         + jnp.dot(sil_ref[...].astype(bf), w1[2 * EMB:3 * EMB], preferred_element_type=jnp.float32)
         + jnp.dot(mat, w1[3 * EMB:4 * EMB], preferred_element_type=jnp.float32)
         + jnp.dot(det, w1[4 * EMB:5 * EMB], preferred_element_type=jnp.float32)
         + b1_ref[...])
    h = jnp.maximum(h, 0.0).astype(bf)
    out = jnp.dot(h, w2_ref[...].astype(bf), preferred_element_type=jnp.float32) + b2_ref[...]
    n = jnp.sqrt(jnp.sum(out * out, axis=-1, keepdims=True))
    n = jnp.maximum(n, 1e-12)
    o_ref[...] = out / n


BR = 1024


def _tc_mlp(cat_e, sil_e, sty_sum, mat_sum, det_sum,
            style_mask, material_mask, detail_mask, W1, b1, W2, b2):
    nb = cat_e.shape[0]
    grid = (nb // BR,)
    return pl.pallas_call(
        _mlp_body,
        grid=grid,
        in_specs=[
            pl.BlockSpec((BR, EMB), lambda i: (i, 0)),
            pl.BlockSpec((BR, EMB), lambda i: (i, 0)),
            pl.BlockSpec((BR, EMB), lambda i: (i, 0)),
            pl.BlockSpec((BR, EMB), lambda i: (i, 0)),
            pl.BlockSpec((BR, EMB), lambda i: (i, 0)),
            pl.BlockSpec((BR, L), lambda i: (i, 0)),
            pl.BlockSpec((BR, L), lambda i: (i, 0)),
            pl.BlockSpec((BR, L), lambda i: (i, 0)),
            pl.BlockSpec((5 * EMB, HID), lambda i: (0, 0)),
            pl.BlockSpec((1, HID), lambda i: (0, 0)),
            pl.BlockSpec((HID, OUT), lambda i: (0, 0)),
            pl.BlockSpec((1, OUT), lambda i: (0, 0)),
        ],
        out_specs=pl.BlockSpec((BR, OUT), lambda i: (i, 0)),
        out_shape=jax.ShapeDtypeStruct((nb, OUT), jnp.float32),
    )(cat_e, sil_e, sty_sum, mat_sum, det_sum,
      style_mask, material_mask, detail_mask, W1, b1, W2, b2)


def _tile_major(idx2d):
    nb = idx2d.shape[0]
    rows_pt = nb // NW
    return idx2d.reshape(NW, rows_pt, L).transpose(0, 2, 1).reshape(-1)


def kernel(category, style, silhouette, material, detail,
           style_mask, material_mask, detail_mask,
           category_table, style_table, silhouette_table,
           material_table, detail_table, W1, b1, W2, b2):
    cat_e, sil_e, sty_sum, mat_sum, det_sum = _sc_gather_pool(
        category, _tile_major(style), silhouette,
        _tile_major(material), _tile_major(detail),
        category_table, style_table, silhouette_table,
        material_table, detail_table)
    return _tc_mlp(cat_e, sil_e, sty_sum, mat_sum, det_sum,
                   style_mask, material_mask, detail_mask,
                   W1, b1.reshape(1, HID), W2, b2.reshape(1, OUT))


# speedup vs baseline: 1.0306x; 1.0306x over previous
import functools

import jax
import jax.numpy as jnp
from jax import lax
from jax.experimental import pallas as pl
from jax.experimental.pallas import tpu as pltpu
from jax.experimental.pallas import tpu_sc as plsc

EMB = 128
HID = 256
OUT = 512
B = 4096
L = 20

NC = 2
NS = 16
NW = NC * NS
LANES = 16


def _sc_gather_pool(category, sty_tl, silhouette, mat_tl, det_tl,
                    cat_t, sty_t, sil_t, mat_t, det_t):
    nb = category.shape[0]
    rows_pt = nb // NW
    pool_pt = nb * L // NW
    mesh = plsc.VectorSubcoreMesh(core_axis_name="c", subcore_axis_name="s")
    out_type = tuple(jax.ShapeDtypeStruct((nb, EMB), jnp.float32)
                     for _ in range(5))

    @functools.partial(
        pl.kernel, mesh=mesh, out_type=out_type,
        scratch_types=[
            pltpu.VMEM((pool_pt,), jnp.int32),
            pltpu.VMEM((pool_pt,), jnp.int32),
            pltpu.VMEM((pool_pt,), jnp.int32),
            pltpu.VMEM((rows_pt,), jnp.int32),
            pltpu.VMEM((rows_pt,), jnp.int32),
            pltpu.VMEM((rows_pt, EMB), jnp.float32),
            pltpu.VMEM((rows_pt, EMB), jnp.float32),
            pltpu.VMEM((rows_pt, EMB), jnp.float32),
            pltpu.VMEM((rows_pt, EMB), jnp.float32),
            pltpu.VMEM((rows_pt, EMB), jnp.float32),
            pltpu.SemaphoreType.DMA,
            pltpu.SemaphoreType.DMA,
            pltpu.SemaphoreType.DMA,
            pltpu.SemaphoreType.DMA,
            pltpu.SemaphoreType.DMA,
            pltpu.SemaphoreType.DMA,
            pltpu.SemaphoreType.DMA,
        ],
    )
    def k(cat_i, sty_i, sil_i, mat_i, det_i,
          cat_th, sty_th, sil_th, mat_th, det_th,
          cat_o, sil_o, sty_o, mat_o, det_o,
          ixp0, ixp1, ixp2, ixs0, ixs1,
          acc0, acc1, acc2, small0, small1,
          sem0, sem1, sem2, sem3, sem4, sem5, sem6):
        wid = lax.axis_index("s") * NC + lax.axis_index("c")
        obase = wid * rows_pt
        gbase = wid * pool_pt

        accs = (acc0, acc1, acc2)
        tabs = (sty_th, mat_th, det_th)
        idxs = (ixp0, ixp1, ixp2)
        sems = (sem0, sem1, sem2)
        outs = (sty_o, mat_o, det_o)

        pltpu.sync_copy(sty_i.at[pl.ds(gbase, pool_pt)], ixp0)
        pltpu.sync_copy(cat_i.at[pl.ds(obase, rows_pt)], ixs0)
        pltpu.sync_copy(sil_i.at[pl.ds(obase, rows_pt)], ixs1)
        pltpu.async_copy(mat_i.at[pl.ds(gbase, pool_pt)], ixp1, sem5)
        pltpu.async_copy(det_i.at[pl.ds(gbase, pool_pt)], ixp2, sem6)

        pltpu.async_copy(cat_th.at[ixs0], small0, sem3)
        pltpu.async_copy(sil_th.at[ixs1], small1, sem4)

        def zero(acc):
            @pl.loop(0, rows_pt)
            def _(r):
                for c in range(EMB // LANES):
                    acc[r, pl.ds(c * LANES, LANES)] = jnp.zeros(
                        (LANES,), jnp.float32)

        def fire(f):
            for l in range(L):
                pltpu.async_copy(
                    tabs[f].at[idxs[f].at[pl.ds(l * rows_pt, rows_pt)]],
                    accs[f], sems[f], add=True)

        zero(acc0)
        fire(0)
        pltpu.make_async_copy(mat_i.at[pl.ds(gbase, pool_pt)], ixp1, sem5).wait()
        zero(acc1)
        fire(1)
        pltpu.make_async_copy(det_i.at[pl.ds(gbase, pool_pt)], ixp2, sem6).wait()
        zero(acc2)
        fire(2)

        for f in range(3):
            for l in range(L):
                pltpu.make_async_copy(
                    tabs[f].at[idxs[f].at[pl.ds(l * rows_pt, rows_pt)]],
                    accs[f], sems[f]).wait()
            pltpu.async_copy(accs[f], outs[f].at[pl.ds(obase, rows_pt)], sems[f])

        pltpu.make_async_copy(cat_th.at[ixs0], small0, sem3).wait()
        pltpu.async_copy(small0, cat_o.at[pl.ds(obase, rows_pt)], sem3)
        pltpu.make_async_copy(sil_th.at[ixs1], small1, sem4).wait()
        pltpu.async_copy(small1, sil_o.at[pl.ds(obase, rows_pt)], sem4)

        for f in range(3):
            pltpu.make_async_copy(accs[f], outs[f].at[pl.ds(obase, rows_pt)],
                                  sems[f]).wait()
        pltpu.make_async_copy(small0, cat_o.at[pl.ds(obase, rows_pt)], sem3).wait()
        pltpu.make_async_copy(small1, sil_o.at[pl.ds(obase, rows_pt)], sem4).wait()

    return k(category, sty_tl, silhouette, mat_tl, det_tl,
             cat_t, sty_t, sil_t, mat_t, det_t)


def _mlp_body(cat_ref, sil_ref, sty_ref, mat_ref, det_ref,
              sm_ref, mm_ref, dm_ref,
              w1_ref, b1_ref, w2_ref, b2_ref, o_ref):
    def pool(sum_ref, m_ref):
        cnt = jnp.maximum(jnp.sum(m_ref[...], axis=1, keepdims=True), 1.0)
        return sum_ref[...] / cnt

    sty = pool(sty_ref, sm_ref)
    mat = pool(mat_ref, mm_ref)
    det = pool(det_ref, dm_ref)
    w1 = w1_ref[...]
    h = (jnp.dot(cat_ref[...], w1[0 * EMB:1 * EMB], preferred_element_type=jnp.float32)
         + jnp.dot(sty, w1[1 * EMB:2 * EMB], preferred_element_type=jnp.float32)
         + jnp.dot(sil_ref[...], w1[2 * EMB:3 * EMB], preferred_element_type=jnp.float32)
         + jnp.dot(mat, w1[3 * EMB:4 * EMB], preferred_element_type=jnp.float32)
         + jnp.dot(det, w1[4 * EMB:5 * EMB], preferred_element_type=jnp.float32)
         + b1_ref[...])
    h = jnp.maximum(h, 0.0)
    out = jnp.dot(h, w2_ref[...], preferred_element_type=jnp.float32) + b2_ref[...]
    n = jnp.sqrt(jnp.sum(out * out, axis=-1, keepdims=True))
    n = jnp.maximum(n, 1e-12)
    o_ref[...] = out / n


BR = 2048


def _tc_mlp(cat_e, sil_e, sty_sum, mat_sum, det_sum,
            style_mask, material_mask, detail_mask, W1, b1, W2, b2):
    nb = cat_e.shape[0]
    grid = (nb // BR,)
    return pl.pallas_call(
        _mlp_body,
        grid=grid,
        in_specs=[
            pl.BlockSpec((BR, EMB), lambda i: (i, 0)),
            pl.BlockSpec((BR, EMB), lambda i: (i, 0)),
            pl.BlockSpec((BR, EMB), lambda i: (i, 0)),
            pl.BlockSpec((BR, EMB), lambda i: (i, 0)),
            pl.BlockSpec((BR, EMB), lambda i: (i, 0)),
            pl.BlockSpec((BR, L), lambda i: (i, 0)),
            pl.BlockSpec((BR, L), lambda i: (i, 0)),
            pl.BlockSpec((BR, L), lambda i: (i, 0)),
            pl.BlockSpec((5 * EMB, HID), lambda i: (0, 0)),
            pl.BlockSpec((1, HID), lambda i: (0, 0)),
            pl.BlockSpec((HID, OUT), lambda i: (0, 0)),
            pl.BlockSpec((1, OUT), lambda i: (0, 0)),
        ],
        out_specs=pl.BlockSpec((BR, OUT), lambda i: (i, 0)),
        out_shape=jax.ShapeDtypeStruct((nb, OUT), jnp.float32),
    )(cat_e, sil_e, sty_sum, mat_sum, det_sum,
      style_mask, material_mask, detail_mask, W1, b1, W2, b2)


def _tile_major(idx2d):
    nb = idx2d.shape[0]
    rows_pt = nb // NW
    return idx2d.reshape(NW, rows_pt, L).transpose(0, 2, 1).reshape(-1)


def kernel(category, style, silhouette, material, detail,
           style_mask, material_mask, detail_mask,
           category_table, style_table, silhouette_table,
           material_table, detail_table, W1, b1, W2, b2):
    cat_e, sil_e, sty_sum, mat_sum, det_sum = _sc_gather_pool(
        category, _tile_major(style), silhouette,
        _tile_major(material), _tile_major(detail),
        category_table, style_table, silhouette_table,
        material_table, detail_table)
    return _tc_mlp(cat_e, sil_e, sty_sum, mat_sum, det_sum,
                   style_mask, material_mask, detail_mask,
                   W1, b1.reshape(1, HID), W2, b2.reshape(1, OUT))
